# baseline (speedup 1.0000x reference)
import jax
import jax.numpy as jnp
from jax import lax
from jax.experimental import pallas as pl
from jax.experimental.pallas import tpu as pltpu
from jax.experimental.pallas import tpu_sc as plsc

_B = 16384
_EMB = 64
_L = 16
_NW = 32
_BW = _B // _NW
_CH = 128
_NCH = _BW // _CH
_GRP = _CH // _L


def _sc_body(users_ref, items_ref, utab, itab, wref, bref, out_ref,
             uidx, iidx, ub0, ub1, ib0, ib1, psc, outb, wbuf, bbuf,
             sem0, sem1):
    wid = lax.axis_index("s") * 2 + lax.axis_index("c")
    base_chunk = wid * _NCH

    pltpu.sync_copy(users_ref.at[pl.ds(base_chunk, _NCH)], uidx)
    pltpu.sync_copy(items_ref.at[pl.ds(base_chunk, _NCH)], iidx)
    pltpu.sync_copy(wref, wbuf)
    pltpu.sync_copy(bref, bbuf)

    wv = [wbuf[j] for j in range(2 * _EMB // _L)]
    bv = bbuf[...]
    iota = lax.iota(jnp.int32, _L)

    ubufs = [ub0, ub1]
    ibufs = [ib0, ib1]
    sems = [sem0, sem1]

    def fire(c):
        s = sems[c % 2]
        du = pltpu.async_copy(utab.at[uidx.at[c]], ubufs[c % 2], s)
        di = pltpu.async_copy(itab.at[iidx.at[c]], ibufs[c % 2], s)
        return du, di

    pending = fire(0)
    for c in range(_NCH):
        nxt = fire(c + 1) if c + 1 < _NCH else None
        du, di = pending
        du.wait()
        di.wait()
        ub = ubufs[c % 2]
        ib = ibufs[c % 2]

        def group(g, carry):
            rowb = g * _L
            for r in range(_L):
                row = rowb + r
                u0 = ub[row, pl.ds(0, _L)]
                u1 = ub[row, pl.ds(_L, _L)]
                u2 = ub[row, pl.ds(2 * _L, _L)]
                u3 = ub[row, pl.ds(3 * _L, _L)]
                i0 = ib[row, pl.ds(0, _L)]
                i1 = ib[row, pl.ds(_L, _L)]
                i2 = ib[row, pl.ds(2 * _L, _L)]
                i3 = ib[row, pl.ds(3 * _L, _L)]
                p = ((u0 * wv[0] + u1 * wv[1]) + (u2 * wv[2] + u3 * wv[3])
                     + (i0 * wv[4] + i1 * wv[5]) + (i2 * wv[6] + i3 * wv[7]))
                psc[r] = p
            acc = bv
            for d in range(_L):
                col = plsc.load_gather(
                    psc, [iota, jnp.full((_L,), d, jnp.int32)])
                acc = acc + col
            outb[pl.ds(c * _CH + rowb, _L)] = acc
            return carry

        lax.fori_loop(0, _GRP, group, 0)
        pending = nxt

    pltpu.sync_copy(outb, out_ref.at[pl.ds(wid * _BW, _BW)])


def kernel(users, items, user_table, item_table, W, b):
    users2d = users.astype(jnp.int32).reshape(_B // _CH, _CH)
    items2d = items.astype(jnp.int32).reshape(_B // _CH, _CH)
    w8 = W.reshape(2 * _EMB // _L, _L)
    b16 = jnp.broadcast_to(b, (_L,))

    mesh = plsc.VectorSubcoreMesh(core_axis_name="c", subcore_axis_name="s")
    f = pl.kernel(
        _sc_body,
        out_type=jax.ShapeDtypeStruct((_B,), jnp.float32),
        mesh=mesh,
        compiler_params=pltpu.CompilerParams(
            needs_layout_passes=False, use_tc_tiling_on_sc=False),
        scratch_types=[
            pltpu.VMEM((_NCH, _CH), jnp.int32),
            pltpu.VMEM((_NCH, _CH), jnp.int32),
            pltpu.VMEM((_CH, _EMB), jnp.float32),
            pltpu.VMEM((_CH, _EMB), jnp.float32),
            pltpu.VMEM((_CH, _EMB), jnp.float32),
            pltpu.VMEM((_CH, _EMB), jnp.float32),
            pltpu.VMEM((_L, _L), jnp.float32),
            pltpu.VMEM((_BW,), jnp.float32),
            pltpu.VMEM((2 * _EMB // _L, _L), jnp.float32),
            pltpu.VMEM((_L,), jnp.float32),
            pltpu.SemaphoreType.DMA,
            pltpu.SemaphoreType.DMA,
        ],
    )
    out = f(users2d, items2d, user_table, item_table, w8, b16)
    return out.reshape(_B, 1)
